# baseline (speedup 1.0000x reference)
import jax
import jax.numpy as jnp
from jax import lax
from jax.experimental import pallas as pl
from jax.experimental.pallas import tpu as pltpu
from jax.experimental.pallas import tpu_sc as plsc

N = 10000
D = 128
E = 320000
L = 4

LANES = 16
NC = 2
NS = 16
NW = NC * NS
B = 128
ROWS = E // B
RP = 2560
RPW = RP // NW
HB = 40
CORE0_H = 3
CORE1_H = 1
NP = 10112
NPT = NP // NS


def _seg_body(x_hbm, src_hbm, dst_hbm, ew_hbm, out_hbm,
              src_a, dst_a, ew_a, rows0, rows1,
              acc_sh, sg0, sg1, ss0, ss1):
    cid = lax.axis_index("c")
    sid = lax.axis_index("s")

    zero = jnp.zeros((LANES,), jnp.float32)

    def _zrow(r, c):
        for j in range(D // LANES):
            rows0[r, pl.ds(LANES * j, LANES)] = zero
        return c

    lax.fori_loop(0, B, _zrow, 0)
    base = sid * NPT
    for k in range(NPT // B):
        pltpu.sync_copy(rows0, acc_sh.at[pl.ds(base + B * k, B)])
    rem = NPT - (NPT // B) * B
    if rem:
        pltpu.sync_copy(rows0.at[pl.ds(0, rem)],
                        acc_sh.at[pl.ds(base + (NPT // B) * B, rem)])
    plsc.subcore_barrier()

    def _compute(buf, t):
        def _grp(gi, cc):
            wv = ew_a[t, pl.ds(LANES * gi, LANES)]
            for rr in range(LANES):
                e = gi * LANES + rr
                w = jnp.full((LANES,), wv[rr], jnp.float32)
                for j in range(D // LANES):
                    v = buf[e, pl.ds(LANES * j, LANES)]
                    v = jnp.maximum(v, 0.01 * v) * w
                    buf[e, pl.ds(LANES * j, LANES)] = v
            return cc

        lax.fori_loop(0, B // LANES, _grp, 0)

    nh = jnp.where(cid == 0, CORE0_H, CORE1_H)
    rbase = jnp.where(cid == 0,
                      NS * HB * CORE1_H + sid * (HB * CORE0_H),
                      sid * (HB * CORE1_H))

    def _half(h, hc):
        row0 = rbase + h * HB
        pltpu.sync_copy(src_hbm.at[pl.ds(row0, HB)], src_a)
        pltpu.sync_copy(dst_hbm.at[pl.ds(row0, HB)], dst_a)
        pltpu.sync_copy(ew_hbm.at[pl.ds(row0, HB)], ew_a)

        pltpu.async_copy(x_hbm.at[src_a.at[0]], rows0, sg0)

        def _iter(t2, c):
            t = 2 * t2

            @pl.when(t2 > 0)
            def _():
                pltpu.make_async_copy(rows1, acc_sh.at[dst_a.at[t - 1]],
                                      ss1).wait()

            pltpu.async_copy(x_hbm.at[src_a.at[t + 1]], rows1, sg1)
            pltpu.make_async_copy(x_hbm.at[src_a.at[t]], rows0, sg0).wait()
            _compute(rows0, t)
            pltpu.async_copy(rows0, acc_sh.at[dst_a.at[t]], ss0, add=True)
            pltpu.make_async_copy(x_hbm.at[src_a.at[t + 1]], rows1, sg1).wait()
            _compute(rows1, t + 1)
            pltpu.async_copy(rows1, acc_sh.at[dst_a.at[t + 1]], ss1, add=True)
            pltpu.make_async_copy(rows0, acc_sh.at[dst_a.at[t]], ss0).wait()

            @pl.when(t + 2 < HB)
            def _():
                pltpu.async_copy(x_hbm.at[src_a.at[t + 2]], rows0, sg0)

            return c

        lax.fori_loop(0, HB // 2, _iter, 0)
        pltpu.make_async_copy(rows1, acc_sh.at[dst_a.at[HB - 1]], ss1).wait()
        return hc

    lax.fori_loop(0, nh, _half, 0)

    plsc.subcore_barrier()
    pltpu.sync_copy(acc_sh.at[pl.ds(base, NPT)],
                    out_hbm.at[cid, pl.ds(base, NPT)])


_seg = pl.kernel(
    _seg_body,
    out_type=jax.ShapeDtypeStruct((NC, NP, D), jnp.float32),
    mesh=plsc.VectorSubcoreMesh(core_axis_name="c", subcore_axis_name="s",
                                num_cores=NC, num_subcores=NS),
    scratch_types=[
        pltpu.VMEM((HB, B), jnp.int32),
        pltpu.VMEM((HB, B), jnp.int32),
        pltpu.VMEM((HB, B), jnp.float32),
        pltpu.VMEM((B, D), jnp.float32),
        pltpu.VMEM((B, D), jnp.float32),
        pltpu.VMEM_SHARED((NP, D), jnp.float32),
        pltpu.SemaphoreType.DMA,
        pltpu.SemaphoreType.DMA,
        pltpu.SemaphoreType.DMA,
        pltpu.SemaphoreType.DMA,
    ],
)

BN = 1000


def _tc_body(x_ref, p_ref, ws_ref, wn_ref, b_ref, g_ref, o_ref):
    x = x_ref[...]
    h = jnp.maximum(x, 0.01 * x)
    agg = p_ref[0] + p_ref[1]
    o_ref[...] = (jnp.dot(h, ws_ref[...], preferred_element_type=jnp.float32)
                  + jnp.dot(agg, wn_ref[...], preferred_element_type=jnp.float32)
                  + b_ref[...] + g_ref[0] * x)


_tc = pl.pallas_call(
    _tc_body,
    grid=(N // BN,),
    in_specs=[
        pl.BlockSpec((BN, D), lambda i: (i, 0)),
        pl.BlockSpec((NC, BN, D), lambda i: (0, i, 0)),
        pl.BlockSpec((D, D), lambda i: (0, 0)),
        pl.BlockSpec((D, D), lambda i: (0, 0)),
        pl.BlockSpec((1, D), lambda i: (0, 0)),
        pl.BlockSpec((1, 1), lambda i: (0, 0)),
    ],
    out_specs=pl.BlockSpec((BN, D), lambda i: (i, 0)),
    out_shape=jax.ShapeDtypeStruct((N, D), jnp.float32),
)


def _pad_rows(a2):
    return jnp.pad(a2, ((0, RP - ROWS), (0, 0)))


def kernel(x, edge_index, edge_weight, W_self, W_neigh, b, gates):
    g = jax.nn.sigmoid(gates)
    srcp = _pad_rows(edge_index[0].reshape(ROWS, B))
    npad = RP - ROWS
    padv = (N + (jnp.arange(npad * B, dtype=jnp.int32) % (NP - N))
            ).reshape(npad, B)
    dstp = jnp.concatenate([edge_index[1].reshape(ROWS, B), padv], axis=0)
    ewp = _pad_rows(edge_weight.reshape(ROWS, B))
    for i in range(L):
        part = _seg(x, srcp, dstp, ewp)
        gi = g[i]
        x = _tc(x, part,
                (1.0 - gi) * W_self[i], (1.0 - gi) * W_neigh[i],
                ((1.0 - gi) * b[i]).reshape(1, D), gi.reshape(1, 1))
    return x


# speedup vs baseline: 3.5025x; 3.5025x over previous
import jax
import jax.numpy as jnp
from jax import lax
from jax.experimental import pallas as pl
from jax.experimental.pallas import tpu as pltpu
from jax.experimental.pallas import tpu_sc as plsc

N = 10000
D = 128
E = 320000
L = 4

LANES = 16
NC = 2
NS = 16
NW = NC * NS
B = 128
ROWS = E // B
RP = 2560
RPW = RP // NW
HB = 40
CORE0_H = 2
CORE1_H = 2
NP = 10112
NPT = NP // NS


def _seg_body(x_hbm, src_hbm, dst_hbm, ew_hbm, out_hbm,
              src_a, dst_a, ew_a, rows0, rows1,
              acc_sh, sg0, sg1, ss0, ss1):
    cid = lax.axis_index("c")
    sid = lax.axis_index("s")

    zero = jnp.zeros((LANES,), jnp.float32)

    def _zrow(r, c):
        for j in range(D // LANES):
            rows0[r, pl.ds(LANES * j, LANES)] = zero
        return c

    lax.fori_loop(0, B, _zrow, 0)
    base = sid * NPT
    for k in range(NPT // B):
        pltpu.sync_copy(rows0, acc_sh.at[pl.ds(base + B * k, B)])
    rem = NPT - (NPT // B) * B
    if rem:
        pltpu.sync_copy(rows0.at[pl.ds(0, rem)],
                        acc_sh.at[pl.ds(base + (NPT // B) * B, rem)])
    plsc.subcore_barrier()

    def _compute(buf, t):
        def _grp(gi, cc):
            wv = ew_a[t, pl.ds(LANES * gi, LANES)]
            for rr in range(LANES):
                e = gi * LANES + rr
                w = jnp.full((LANES,), wv[rr], jnp.float32)
                for j in range(D // LANES):
                    v = buf[e, pl.ds(LANES * j, LANES)]
                    v = jnp.maximum(v, 0.01 * v) * w
                    buf[e, pl.ds(LANES * j, LANES)] = v
            return cc

        lax.fori_loop(0, B // LANES, _grp, 0)

    nh = jnp.where(cid == 0, CORE0_H, CORE1_H)
    rbase = jnp.where(cid == 0,
                      NS * HB * CORE1_H + sid * (HB * CORE0_H),
                      sid * (HB * CORE1_H))

    def _half(h, hc):
        row0 = rbase + h * HB
        pltpu.sync_copy(src_hbm.at[pl.ds(row0, HB)], src_a)
        pltpu.sync_copy(dst_hbm.at[pl.ds(row0, HB)], dst_a)
        pltpu.sync_copy(ew_hbm.at[pl.ds(row0, HB)], ew_a)

        pltpu.async_copy(x_hbm.at[src_a.at[0]], rows0, sg0)

        def _iter(t2, c):
            t = 2 * t2

            @pl.when(t2 > 0)
            def _():
                pltpu.make_async_copy(rows1, acc_sh.at[dst_a.at[t - 1]],
                                      ss1).wait()

            pltpu.async_copy(x_hbm.at[src_a.at[t + 1]], rows1, sg1)
            pltpu.make_async_copy(x_hbm.at[src_a.at[t]], rows0, sg0).wait()
            _compute(rows0, t)
            pltpu.async_copy(rows0, acc_sh.at[dst_a.at[t]], ss0, add=True)
            pltpu.make_async_copy(x_hbm.at[src_a.at[t + 1]], rows1, sg1).wait()
            _compute(rows1, t + 1)
            pltpu.async_copy(rows1, acc_sh.at[dst_a.at[t + 1]], ss1, add=True)
            pltpu.make_async_copy(rows0, acc_sh.at[dst_a.at[t]], ss0).wait()

            @pl.when(t + 2 < HB)
            def _():
                pltpu.async_copy(x_hbm.at[src_a.at[t + 2]], rows0, sg0)

            return c

        lax.fori_loop(0, HB // 2, _iter, 0)
        pltpu.make_async_copy(rows1, acc_sh.at[dst_a.at[HB - 1]], ss1).wait()
        return hc

    lax.fori_loop(0, nh, _half, 0)

    plsc.subcore_barrier()
    pltpu.sync_copy(acc_sh.at[pl.ds(base, NPT)],
                    out_hbm.at[cid, pl.ds(base, NPT)])


_seg = pl.kernel(
    _seg_body,
    out_type=jax.ShapeDtypeStruct((NC, NP, D), jnp.float32),
    mesh=plsc.VectorSubcoreMesh(core_axis_name="c", subcore_axis_name="s",
                                num_cores=NC, num_subcores=NS),
    scratch_types=[
        pltpu.VMEM((HB, B), jnp.int32),
        pltpu.VMEM((HB, B), jnp.int32),
        pltpu.VMEM((HB, B), jnp.float32),
        pltpu.VMEM((B, D), jnp.float32),
        pltpu.VMEM((B, D), jnp.float32),
        pltpu.VMEM_SHARED((NP, D), jnp.float32),
        pltpu.SemaphoreType.DMA,
        pltpu.SemaphoreType.DMA,
        pltpu.SemaphoreType.DMA,
        pltpu.SemaphoreType.DMA,
    ],
)

BN = 1000


def _tc_body(x_ref, p_ref, ws_ref, wn_ref, b_ref, g_ref, o_ref):
    x = x_ref[...]
    h = jnp.maximum(x, 0.01 * x)
    agg = p_ref[0] + p_ref[1]
    o_ref[...] = (jnp.dot(h, ws_ref[...], preferred_element_type=jnp.float32)
                  + jnp.dot(agg, wn_ref[...], preferred_element_type=jnp.float32)
                  + b_ref[...] + g_ref[0] * x)


_tc = pl.pallas_call(
    _tc_body,
    grid=(N // BN,),
    in_specs=[
        pl.BlockSpec((BN, D), lambda i: (i, 0)),
        pl.BlockSpec((NC, BN, D), lambda i: (0, i, 0)),
        pl.BlockSpec((D, D), lambda i: (0, 0)),
        pl.BlockSpec((D, D), lambda i: (0, 0)),
        pl.BlockSpec((1, D), lambda i: (0, 0)),
        pl.BlockSpec((1, 1), lambda i: (0, 0)),
    ],
    out_specs=pl.BlockSpec((BN, D), lambda i: (i, 0)),
    out_shape=jax.ShapeDtypeStruct((N, D), jnp.float32),
)


def _pad_rows(a2):
    return jnp.pad(a2, ((0, RP - ROWS), (0, 0)))


def kernel(x, edge_index, edge_weight, W_self, W_neigh, b, gates):
    g = jax.nn.sigmoid(gates)
    npad = RP - ROWS
    pad_src = (jnp.arange(npad * B, dtype=jnp.int32) % N).reshape(npad, B)
    srcp = jnp.concatenate([edge_index[0].reshape(ROWS, B), pad_src], axis=0)
    pad_dst = (N + (jnp.arange(npad * B, dtype=jnp.int32) % (NP - N))
               ).reshape(npad, B)
    dstp = jnp.concatenate([edge_index[1].reshape(ROWS, B), pad_dst], axis=0)
    ewp = _pad_rows(edge_weight.reshape(ROWS, B))
    for i in range(L):
        part = _seg(x, srcp, dstp, ewp)
        gi = g[i]
        x = _tc(x, part,
                (1.0 - gi) * W_self[i], (1.0 - gi) * W_neigh[i],
                ((1.0 - gi) * b[i]).reshape(1, D), gi.reshape(1, 1))
    return x


# speedup vs baseline: 3.5212x; 1.0053x over previous
import jax
import jax.numpy as jnp
from jax import lax
from jax.experimental import pallas as pl
from jax.experimental.pallas import tpu as pltpu
from jax.experimental.pallas import tpu_sc as plsc

N = 10000
D = 128
E = 320000
L = 4

LANES = 16
NC = 2
NS = 16
NW = NC * NS
B = 128
ROWS = E // B
RP = 2560
RPW = RP // NW
HB = 40
CORE0_H = 2
CORE1_H = 2
NP = 10112
NPT = NP // NS


def _seg_body(x_hbm, src_hbm, dst_hbm, ew_hbm, out_hbm,
              src_a, dst_a, ew_a, rows0, rows1,
              acc_sh, sg0, sg1, ss0, ss1):
    cid = lax.axis_index("c")
    sid = lax.axis_index("s")

    zero = jnp.zeros((LANES,), jnp.float32)

    def _zrow(r, c):
        for j in range(D // LANES):
            rows0[r, pl.ds(LANES * j, LANES)] = zero
        return c

    lax.fori_loop(0, B, _zrow, 0)
    base = sid * NPT
    for k in range(NPT // B):
        pltpu.sync_copy(rows0, acc_sh.at[pl.ds(base + B * k, B)])
    rem = NPT - (NPT // B) * B
    if rem:
        pltpu.sync_copy(rows0.at[pl.ds(0, rem)],
                        acc_sh.at[pl.ds(base + (NPT // B) * B, rem)])
    plsc.subcore_barrier()

    def _compute(buf, t):
        def _grp(gi, cc):
            wv = ew_a[t, pl.ds(LANES * gi, LANES)]
            for rr in range(LANES):
                e = gi * LANES + rr
                w = jnp.full((LANES,), wv[rr], jnp.float32)
                for j in range(D // LANES):
                    v = buf[e, pl.ds(LANES * j, LANES)]
                    v = jnp.maximum(v, 0.01 * v) * w
                    buf[e, pl.ds(LANES * j, LANES)] = v
            return cc

        lax.fori_loop(0, B // LANES, _grp, 0)

    nh = jnp.where(cid == 0, CORE0_H, CORE1_H)
    rbase = jnp.where(cid == 0,
                      NS * HB * CORE1_H + sid * (HB * CORE0_H),
                      sid * (HB * CORE1_H))

    def _half(h, hc):
        row0 = rbase + h * HB
        pltpu.sync_copy(src_hbm.at[pl.ds(row0, HB)], src_a)
        pltpu.sync_copy(dst_hbm.at[pl.ds(row0, HB)], dst_a)
        pltpu.sync_copy(ew_hbm.at[pl.ds(row0, HB)], ew_a)

        pltpu.async_copy(x_hbm.at[src_a.at[0]], rows0, sg0)

        def _iter(t2, c):
            t = 2 * t2

            @pl.when(t2 > 0)
            def _():
                pltpu.make_async_copy(rows1, acc_sh.at[dst_a.at[t - 1]],
                                      ss1).wait()

            pltpu.async_copy(x_hbm.at[src_a.at[t + 1]], rows1, sg1)
            pltpu.make_async_copy(x_hbm.at[src_a.at[t]], rows0, sg0).wait()
            _compute(rows0, t)
            pltpu.async_copy(rows0, acc_sh.at[dst_a.at[t]], ss0, add=True)
            pltpu.make_async_copy(x_hbm.at[src_a.at[t + 1]], rows1, sg1).wait()
            _compute(rows1, t + 1)
            pltpu.async_copy(rows1, acc_sh.at[dst_a.at[t + 1]], ss1, add=True)
            pltpu.make_async_copy(rows0, acc_sh.at[dst_a.at[t]], ss0).wait()

            @pl.when(t + 2 < HB)
            def _():
                pltpu.async_copy(x_hbm.at[src_a.at[t + 2]], rows0, sg0)

            return c

        lax.fori_loop(0, HB // 2, _iter, 0)
        pltpu.make_async_copy(rows1, acc_sh.at[dst_a.at[HB - 1]], ss1).wait()
        return hc

    lax.fori_loop(0, nh, _half, 0)

    plsc.subcore_barrier()
    pltpu.sync_copy(acc_sh.at[pl.ds(base, NPT)],
                    out_hbm.at[cid, pl.ds(base, NPT)])


_seg = pl.kernel(
    _seg_body,
    out_type=jax.ShapeDtypeStruct((NC, NP, D), jnp.float32),
    mesh=plsc.VectorSubcoreMesh(core_axis_name="c", subcore_axis_name="s",
                                num_cores=NC, num_subcores=NS),
    scratch_types=[
        pltpu.VMEM((HB, B), jnp.int32),
        pltpu.VMEM((HB, B), jnp.int32),
        pltpu.VMEM((HB, B), jnp.float32),
        pltpu.VMEM((B, D), jnp.float32),
        pltpu.VMEM((B, D), jnp.float32),
        pltpu.VMEM_SHARED((NP, D), jnp.float32),
        pltpu.SemaphoreType.DMA,
        pltpu.SemaphoreType.DMA,
        pltpu.SemaphoreType.DMA,
        pltpu.SemaphoreType.DMA,
    ],
)

BN = 1000


def _tc_self_body(x_ref, ws_ref, b_ref, g_ref, o_ref):
    x = x_ref[...]
    h = jnp.maximum(x, 0.01 * x)
    o_ref[...] = (jnp.dot(h, ws_ref[...], preferred_element_type=jnp.float32)
                  + b_ref[...] + g_ref[0] * x)


_tc_self = pl.pallas_call(
    _tc_self_body,
    grid=(N // BN,),
    in_specs=[
        pl.BlockSpec((BN, D), lambda i: (i, 0)),
        pl.BlockSpec((D, D), lambda i: (0, 0)),
        pl.BlockSpec((1, D), lambda i: (0, 0)),
        pl.BlockSpec((1, 1), lambda i: (0, 0)),
    ],
    out_specs=pl.BlockSpec((BN, D), lambda i: (i, 0)),
    out_shape=jax.ShapeDtypeStruct((N, D), jnp.float32),
)


def _tc_neigh_body(s_ref, p_ref, wn_ref, o_ref):
    agg = p_ref[0] + p_ref[1]
    o_ref[...] = s_ref[...] + jnp.dot(agg, wn_ref[...],
                                      preferred_element_type=jnp.float32)


_tc_neigh = pl.pallas_call(
    _tc_neigh_body,
    grid=(N // BN,),
    in_specs=[
        pl.BlockSpec((BN, D), lambda i: (i, 0)),
        pl.BlockSpec((NC, BN, D), lambda i: (0, i, 0)),
        pl.BlockSpec((D, D), lambda i: (0, 0)),
    ],
    out_specs=pl.BlockSpec((BN, D), lambda i: (i, 0)),
    out_shape=jax.ShapeDtypeStruct((N, D), jnp.float32),
)


def _pad_rows(a2):
    return jnp.pad(a2, ((0, RP - ROWS), (0, 0)))


def kernel(x, edge_index, edge_weight, W_self, W_neigh, b, gates):
    g = jax.nn.sigmoid(gates)
    npad = RP - ROWS
    pad_src = (jnp.arange(npad * B, dtype=jnp.int32) % N).reshape(npad, B)
    srcp = jnp.concatenate([edge_index[0].reshape(ROWS, B), pad_src], axis=0)
    pad_dst = (N + (jnp.arange(npad * B, dtype=jnp.int32) % (NP - N))
               ).reshape(npad, B)
    dstp = jnp.concatenate([edge_index[1].reshape(ROWS, B), pad_dst], axis=0)
    ewp = _pad_rows(edge_weight.reshape(ROWS, B))
    for i in range(L):
        part = _seg(x, srcp, dstp, ewp)
        gi = g[i]
        s = _tc_self(x, (1.0 - gi) * W_self[i],
                     ((1.0 - gi) * b[i]).reshape(1, D), gi.reshape(1, 1))
        x = _tc_neigh(s, part, (1.0 - gi) * W_neigh[i])
    return x
